# baseline (speedup 1.0000x reference)
import jax
import jax.numpy as jnp
from jax.experimental import pallas as pl
from jax.experimental.pallas import tpu as pltpu
from jax.experimental.pallas import tpu_sc as plsc

_TW = 32768


def _tc_pack(tabT, interpret=False):
    nblk = -(-(_F * _V) // _TW)
    rows = nblk * (_TW // 2)
    grid = (nblk,)

    def body(x_ref, o_ref):
        a = x_ref[:, : _TW // 2].T
        b = x_ref[:, _TW // 2:].T
        o_ref[...] = jnp.concatenate([a, b], axis=1)

    return pl.pallas_call(
        body,
        grid=grid,
        in_specs=[pl.BlockSpec((_D, _TW), lambda i: (0, i))],
        out_specs=pl.BlockSpec((_TW // 2, 2 * _D), lambda i: (i, 0)),
        out_shape=jax.ShapeDtypeStruct((rows, 2 * _D), jnp.float32),
        compiler_params=pltpu.CompilerParams(
            dimension_semantics=("parallel",)),
        interpret=interpret,
    )(tabT)

_B = 4096
_F = 26
_V = 100000
_D = 64
_NPAIR = (_F + 1) * _F // 2
_ZROWS = 352


def _sc_gather(packed_table, pair_idx):
    n = _B * _F
    win = 128
    mesh = plsc.VectorSubcoreMesh(core_axis_name="c", subcore_axis_name="s")

    @pl.kernel(
        out_type=jax.ShapeDtypeStruct((n, 2 * _D), packed_table.dtype),
        mesh=mesh,
    )
    def gather_kernel(tab_hbm, idx_hbm, out_hbm):
        def body(i_vmem, o_vmem):
            pltpu.sync_copy(tab_hbm.at[i_vmem.at[0]], o_vmem)

        pltpu.emit_pipeline(
            body,
            grid=(n // win,),
            in_specs=[pl.BlockSpec((1, win), index_map=lambda i: (0, i))],
            out_specs=[pl.BlockSpec((win, 2 * _D), index_map=lambda i: (i, 0))],
            core_axis_name=("c", "s"),
            dimension_semantics=(pltpu.PARALLEL,),
        )(idx_hbm, out_hbm)

    return gather_kernel(packed_table, pair_idx)


def _tc_body(denseT_ref, emb_ref, par_ref,
             dW0T_ref, db0_ref, dW1T_ref, db1_ref, dW2T_ref, db2_ref,
             oW0aT_ref, oW0iT_ref, ob0_ref, oW1T_ref, ob1_ref,
             oW2T_ref, ob2_ref,
             out_ref, ct_ref, z_ref):
    f32 = jnp.float32
    h = jnp.maximum(jnp.dot(dW0T_ref[...], denseT_ref[...],
                            preferred_element_type=f32) + db0_ref[...], 0.0)
    h = jnp.maximum(jnp.dot(dW1T_ref[...], h,
                            preferred_element_type=f32) + db1_ref[...], 0.0)
    edT = jnp.maximum(jnp.dot(dW2T_ref[...], h,
                              preferred_element_type=f32) + db2_ref[...], 0.0)

    ct_ref[0, :, :] = edT
    for f in range(_F):
        eT = emb_ref[f].T
        pf = par_ref[f:f + 1, :]
        ct_ref[f + 1, :, :] = jnp.where(pf > 0.5, eT[_D:], eT[:_D])

    z_ref[_NPAIR:_ZROWS, :] = jnp.zeros((_ZROWS - _NPAIR, z_ref.shape[1]), f32)
    for i in range(1, _F + 1):
        s = i * (i - 1) // 2
        prod = ct_ref[0:i] * ct_ref[i][None, :, :]
        z_ref[s:s + i, :] = jnp.sum(prod, axis=1)

    h = jnp.dot(oW0aT_ref[...], edT, preferred_element_type=f32)
    h = h + jnp.dot(oW0iT_ref[...], z_ref[...], preferred_element_type=f32)
    h = jnp.maximum(h + ob0_ref[...], 0.0)
    h = jnp.maximum(jnp.dot(oW1T_ref[...], h,
                            preferred_element_type=f32) + ob1_ref[...], 0.0)
    out_ref[...] = jnp.dot(oW2T_ref[...], h,
                           preferred_element_type=f32) + ob2_ref[...]


def _tc_forward(denseT, embF, parity, dW0T, db0, dW1T, db1, dW2T, db2,
                oW0aT, oW0iT, ob0, oW1T, ob1, oW2T, ob2,
                interpret=False):
    Bb = 512
    grid = (_B // Bb,)

    def full(a):
        return pl.BlockSpec(a.shape, lambda i: (0,) * a.ndim)

    in_specs = [
        pl.BlockSpec((13, Bb), lambda i: (0, i)),
        pl.BlockSpec((_F, Bb, 2 * _D), lambda i: (0, i, 0)),
        pl.BlockSpec((_F, Bb), lambda i: (0, i)),
        full(dW0T), full(db0), full(dW1T), full(db1), full(dW2T), full(db2),
        full(oW0aT), full(oW0iT), full(ob0), full(oW1T), full(ob1),
        full(oW2T), full(ob2),
    ]
    return pl.pallas_call(
        _tc_body,
        grid=grid,
        in_specs=in_specs,
        out_specs=pl.BlockSpec((1, Bb), lambda i: (0, i)),
        out_shape=jax.ShapeDtypeStruct((1, _B), jnp.float32),
        scratch_shapes=[
            pltpu.VMEM((_F + 1, _D, Bb), jnp.float32),
            pltpu.VMEM((_ZROWS, Bb), jnp.float32),
        ],
        compiler_params=pltpu.CompilerParams(
            dimension_semantics=("arbitrary",)),
        interpret=interpret,
    )(denseT, embF, parity, dW0T, db0, dW1T, db1, dW2T, db2,
      oW0aT, oW0iT, ob0, oW1T, ob1, oW2T, ob2)


def kernel(dense_features, sparse_features, emb_table,
           dW0, db0, dW1, db1, dW2, db2,
           oW0, ob0, oW1, ob1, oW2, ob2):
    offsets = jnp.arange(_F, dtype=jnp.int32) * _V
    flat_idx = (sparse_features + offsets[None, :]).T
    blk = flat_idx // _TW
    rem = flat_idx % _TW
    half = _TW // 2
    pair_idx = (blk * half + rem % half).reshape(1, _F * _B)
    parity = (rem // half).astype(jnp.float32)

    packed_table = _tc_pack(emb_table.T)
    gathered = _sc_gather(packed_table, pair_idx)
    embF = gathered.reshape(_F, _B, 2 * _D)

    col = lambda b: b.reshape(-1, 1)
    out = _tc_forward(
        dense_features.T, embF, parity,
        dW0.T, col(db0), dW1.T, col(db1), dW2.T, col(db2),
        oW0[:_D].T,
        jnp.concatenate(
            [oW0[_D:].T, jnp.zeros((512, _ZROWS - _NPAIR), jnp.float32)],
            axis=1),
        col(ob0), oW1.T, col(ob1), oW2.T, col(ob2),
    )
    return out.reshape(_B, 1)


# speedup vs baseline: 1.0946x; 1.0946x over previous
import jax
import jax.numpy as jnp
from jax.experimental import pallas as pl
from jax.experimental.pallas import tpu as pltpu
from jax.experimental.pallas import tpu_sc as plsc

_TW = 32768


def _tc_pack(tabT, interpret=False):
    nblk = -(-(_F * _V) // _TW)
    q = _TW // 4
    rows = nblk * q
    grid = (nblk,)

    def body(x_ref, o_ref):
        x16 = x_ref[...].astype(jnp.bfloat16)
        half = _TW // 2
        a = pltpu.bitcast(x16[:, :half].T, jnp.float32)
        b = pltpu.bitcast(x16[:, half:].T, jnp.float32)
        o_ref[...] = jnp.concatenate([a, b], axis=1)

    return pl.pallas_call(
        body,
        grid=grid,
        in_specs=[pl.BlockSpec((_D, _TW), lambda i: (0, i))],
        out_specs=pl.BlockSpec((q, 2 * _D), lambda i: (i, 0)),
        out_shape=jax.ShapeDtypeStruct((rows, 2 * _D), jnp.float32),
        compiler_params=pltpu.CompilerParams(
            dimension_semantics=("parallel",)),
        interpret=interpret,
    )(tabT)

_B = 4096
_F = 26
_V = 100000
_D = 64
_NPAIR = (_F + 1) * _F // 2
_ZROWS = 352


def _sc_gather(packed_table, pair_idx):
    n = _B * _F
    win = 128
    mesh = plsc.VectorSubcoreMesh(core_axis_name="c", subcore_axis_name="s")

    @pl.kernel(
        out_type=jax.ShapeDtypeStruct((n, 2 * _D), packed_table.dtype),
        mesh=mesh,
    )
    def gather_kernel(tab_hbm, idx_hbm, out_hbm):
        def body(i_vmem, o_vmem):
            pltpu.sync_copy(tab_hbm.at[i_vmem.at[0]], o_vmem)

        pltpu.emit_pipeline(
            body,
            grid=(n // win,),
            in_specs=[pl.BlockSpec((1, win), index_map=lambda i: (0, i))],
            out_specs=[pl.BlockSpec((win, 2 * _D), index_map=lambda i: (i, 0))],
            core_axis_name=("c", "s"),
            dimension_semantics=(pltpu.PARALLEL,),
        )(idx_hbm, out_hbm)

    return gather_kernel(packed_table, pair_idx)


def _tc_body(denseT_ref, emb_ref, parh_ref, pars_ref,
             dW0T_ref, db0_ref, dW1T_ref, db1_ref, dW2T_ref, db2_ref,
             oW0aT_ref, oW0iT_ref, ob0_ref, oW1T_ref, ob1_ref,
             oW2T_ref, ob2_ref,
             out_ref, ct_ref, z_ref):
    f32 = jnp.float32
    h = jnp.maximum(jnp.dot(dW0T_ref[...], denseT_ref[...],
                            preferred_element_type=f32) + db0_ref[...], 0.0)
    h = jnp.maximum(jnp.dot(dW1T_ref[...], h,
                            preferred_element_type=f32) + db1_ref[...], 0.0)
    edT = jnp.maximum(jnp.dot(dW2T_ref[...], h,
                              preferred_element_type=f32) + db2_ref[...], 0.0)

    ct_ref[0, :, :] = edT
    Bb = out_ref.shape[1]
    for f in range(_F):
        e = emb_ref[f]
        hb = parh_ref[:, f:f + 1]
        sub = pars_ref[:, f:f + 1]
        hsel = jnp.where(hb > 0.5, e[:, _D:], e[:, :_D])
        rows16 = pltpu.bitcast(hsel, jnp.bfloat16)
        pairs = rows16.reshape(Bb, 2, _D)
        even = pairs[:, 0:1, :].reshape(Bb, _D).astype(f32)
        odd = pairs[:, 1:2, :].reshape(Bb, _D).astype(f32)
        ct_ref[f + 1, :, :] = jnp.where(sub > 0.5, odd, even).T

    z_ref[_NPAIR:_ZROWS, :] = jnp.zeros((_ZROWS - _NPAIR, z_ref.shape[1]), f32)
    for i in range(1, _F + 1):
        s = i * (i - 1) // 2
        prod = ct_ref[0:i] * ct_ref[i][None, :, :]
        z_ref[s:s + i, :] = jnp.sum(prod, axis=1)

    h = jnp.dot(oW0aT_ref[...], edT, preferred_element_type=f32)
    h = h + jnp.dot(oW0iT_ref[...], z_ref[...], preferred_element_type=f32)
    h = jnp.maximum(h + ob0_ref[...], 0.0)
    h = jnp.maximum(jnp.dot(oW1T_ref[...], h,
                            preferred_element_type=f32) + ob1_ref[...], 0.0)
    out_ref[...] = jnp.dot(oW2T_ref[...], h,
                           preferred_element_type=f32) + ob2_ref[...]


def _tc_forward(denseT, embF, parh, pars, dW0T, db0, dW1T, db1, dW2T, db2,
                oW0aT, oW0iT, ob0, oW1T, ob1, oW2T, ob2,
                interpret=False):
    Bb = 512
    grid = (_B // Bb,)

    def full(a):
        return pl.BlockSpec(a.shape, lambda i: (0,) * a.ndim)

    in_specs = [
        pl.BlockSpec((13, Bb), lambda i: (0, i)),
        pl.BlockSpec((_F, Bb, 2 * _D), lambda i: (0, i, 0)),
        pl.BlockSpec((Bb, _F), lambda i: (i, 0)),
        pl.BlockSpec((Bb, _F), lambda i: (i, 0)),
        full(dW0T), full(db0), full(dW1T), full(db1), full(dW2T), full(db2),
        full(oW0aT), full(oW0iT), full(ob0), full(oW1T), full(ob1),
        full(oW2T), full(ob2),
    ]
    return pl.pallas_call(
        _tc_body,
        grid=grid,
        in_specs=in_specs,
        out_specs=pl.BlockSpec((1, Bb), lambda i: (0, i)),
        out_shape=jax.ShapeDtypeStruct((1, _B), jnp.float32),
        scratch_shapes=[
            pltpu.VMEM((_F + 1, _D, Bb), jnp.float32),
            pltpu.VMEM((_ZROWS, Bb), jnp.float32),
        ],
        compiler_params=pltpu.CompilerParams(
            dimension_semantics=("arbitrary",)),
        interpret=interpret,
    )(denseT, embF, parh, pars, dW0T, db0, dW1T, db1, dW2T, db2,
      oW0aT, oW0iT, ob0, oW1T, ob1, oW2T, ob2)


def kernel(dense_features, sparse_features, emb_table,
           dW0, db0, dW1, db1, dW2, db2,
           oW0, ob0, oW1, ob1, oW2, ob2):
    offsets = jnp.arange(_F, dtype=jnp.int32) * _V
    flat_idx = (sparse_features + offsets[None, :]).T
    blk = flat_idx // _TW
    rem = flat_idx % _TW
    half = _TW // 2
    hb = rem // half
    hr = rem % half
    pair_idx = (blk * (half // 2) + hr // 2).reshape(1, _F * _B)
    parh = hb.astype(jnp.float32).T
    pars = (hr % 2).astype(jnp.float32).T

    packed_table = _tc_pack(emb_table.T)
    gathered = _sc_gather(packed_table, pair_idx)
    embF = gathered.reshape(_F, _B, 2 * _D)

    col = lambda b: b.reshape(-1, 1)
    out = _tc_forward(
        dense_features.T, embF, parh, pars,
        dW0.T, col(db0), dW1.T, col(db1), dW2.T, col(db2),
        oW0[:_D].T,
        jnp.concatenate(
            [oW0[_D:].T, jnp.zeros((512, _ZROWS - _NPAIR), jnp.float32)],
            axis=1),
        col(ob0), oW1.T, col(ob1), oW2.T, col(ob2),
    )
    return out.reshape(_B, 1)
